# Initial kernel scaffold; baseline (speedup 1.0000x reference)
import jax
import jax.numpy as jnp
from jax.experimental import pallas as pl


def kernel(x, params, edge_index, batch):
    raise NotImplementedError("write your pallas kernel here")


# speedup vs baseline: 1.0051x; 1.0051x over previous
import functools

import jax
import jax.numpy as jnp
from jax import lax
from jax.experimental import pallas as pl
from jax.experimental.pallas import tpu as pltpu, tpu_sc as plsc

N = 50000
E = 800000
F = 74
FP = 80
H = 256
G = 512
OUT = 2
EPS = 1e-5

TE = 3200
TN = 2000
GE = E // TE
GN = N // TN

NC, NS = 2, 16
NW = NC * NS
EW = E // NW
GCH = 1000
RNG = 2048
NRANGES = 26
NPAD = RNG * NRANGES
RPAD = 16
SCH = 10000
ECT = E // NS
DK = 128

_mesh = plsc.VectorSubcoreMesh(core_axis_name="c", subcore_axis_name="s")


@functools.partial(
    pl.kernel, mesh=_mesh,
    out_type=(jax.ShapeDtypeStruct((E, FP), jnp.float32),
              jax.ShapeDtypeStruct((E, FP), jnp.float32)),
    scratch_types=[pltpu.VMEM((GCH,), jnp.int32),
                   pltpu.VMEM((GCH, FP), jnp.float32),
                   pltpu.SemaphoreType.DMA],
    compiler_params=pltpu.CompilerParams(use_tc_tiling_on_sc=False),
)
def _sc_gather(xp_hbm, row_hbm, col_hbm, ar_hbm, ac_hbm, idx_v, buf, sem):
    wid = lax.axis_index("s") * NC + lax.axis_index("c")
    base = wid * EW

    def step(i, _):
        off = base + i * GCH
        pltpu.sync_copy(row_hbm.at[pl.ds(off, GCH)], idx_v)
        pltpu.async_copy(xp_hbm.at[idx_v], buf, sem).wait()
        pltpu.sync_copy(buf, ar_hbm.at[pl.ds(off, GCH)])
        pltpu.sync_copy(col_hbm.at[pl.ds(off, GCH)], idx_v)
        pltpu.async_copy(xp_hbm.at[idx_v], buf, sem).wait()
        pltpu.sync_copy(buf, ac_hbm.at[pl.ds(off, GCH)])
        return _

    lax.fori_loop(0, EW // GCH, step, 0)


@functools.partial(
    pl.kernel, mesh=_mesh,
    out_type=(jax.ShapeDtypeStruct((NPAD, H), jnp.float32),
              jax.ShapeDtypeStruct((NPAD, 16), jnp.float32)),
    scratch_types=[pltpu.VMEM((SCH,), jnp.int32),
                   pltpu.VMEM((SCH + 160,), jnp.int32),
                   pltpu.VMEM((SCH + 160,), jnp.int32),
                   pltpu.VMEM((DK,), jnp.int32),
                   pltpu.VMEM((DK, H), jnp.float32),
                   pltpu.VMEM((64, H), jnp.float32),
                   pltpu.VMEM(((RNG + RPAD) // NS, 16), jnp.float32),
                   pltpu.VMEM((DK, 16), jnp.float32),
                   pltpu.VMEM_SHARED((RNG + RPAD, H), jnp.float32),
                   pltpu.VMEM_SHARED((RNG + RPAD, 16), jnp.float32),
                   pltpu.SemaphoreType.DMA],
    compiler_params=pltpu.CompilerParams(use_tc_tiling_on_sc=False,
                                         needs_layout_passes=False),
)
def _sc_scatter(g2_hbm, col_hbm, z_hbm, z16_hbm, on_hbm,
                nsum_hbm, ncnt_hbm,
                cbuf, midx, mloc, loc_buf, dbuf, zbuf, z16v, ones_v,
                acc, cnta, sem):
    c = lax.axis_index("c")
    s = lax.axis_index("s")
    iota = lax.iota(jnp.int32, 16)
    tbase = s * ECT
    pltpu.sync_copy(z_hbm, zbuf)
    pltpu.sync_copy(z16_hbm, z16v)
    pltpu.sync_copy(on_hbm, ones_v)

    def range_step(r_i, _):
        rng = 2 * r_i + c
        lo = rng * RNG
        def zstep(k, _z):
            pltpu.sync_copy(zbuf,
                            acc.at[pl.ds(s * (RNG // NS) + k * 64, 64)])
            return _z
        lax.fori_loop(0, RNG // NS // 64, zstep, 0)
        pltpu.sync_copy(z16v, cnta.at[pl.ds(s * ((RNG + RPAD) // NS),
                                            (RNG + RPAD) // NS)])
        plsc.subcore_barrier()

        def chunk_step(ch, _c):
            ebase = tbase + ch * SCH
            pltpu.sync_copy(col_hbm.at[pl.ds(ebase, SCH)], cbuf)

            def scan_step(v, cur):
                cols = cbuf[pl.ds(v * 16, 16)]
                m = (cols >= lo) & (cols < lo + RNG)
                pre = plsc.cumsum(m.astype(jnp.int32))
                eid = ebase + v * 16 + iota
                dst = jnp.where(m, cur + pre - 1, SCH + 128 + iota)
                plsc.store_scatter(midx, [dst], eid)
                plsc.store_scatter(mloc, [dst], cols - lo)
                return cur + jnp.max(pre)

            mcnt = lax.fori_loop(0, SCH // 16, scan_step, 0)
            for j in range(DK // 16):
                midx[pl.ds(mcnt + j * 16, 16)] = jnp.zeros((16,), jnp.int32)
                mloc[pl.ds(mcnt + j * 16, 16)] = jnp.full((16,), RNG,
                                                          jnp.int32)

            def drain_step(k, _d):
                for j in range(DK // 16):
                    loc_buf[pl.ds(j * 16, 16)] = mloc[pl.ds(k * DK + j * 16,
                                                            16)]
                pltpu.async_copy(g2_hbm.at[midx.at[pl.ds(k * DK, DK)]],
                                 dbuf, sem).wait()
                pltpu.sync_copy(dbuf, acc.at[loc_buf], add=True)
                pltpu.sync_copy(ones_v, cnta.at[loc_buf], add=True)
                return _d

            lax.fori_loop(0, (mcnt + DK - 1) // DK, drain_step, 0)
            return _c

        lax.fori_loop(0, ECT // SCH, chunk_step, 0)
        plsc.subcore_barrier()

        out0 = rng * RNG + s * (RNG // NS)
        pltpu.sync_copy(acc.at[pl.ds(s * (RNG // NS), RNG // NS)],
                        nsum_hbm.at[pl.ds(out0, RNG // NS)])
        pltpu.sync_copy(cnta.at[pl.ds(s * (RNG // NS), RNG // NS)],
                        ncnt_hbm.at[pl.ds(out0, RNG // NS)])
        plsc.subcore_barrier()
        return _

    lax.fori_loop(0, NRANGES // NC, range_step, 0)


def _full(shape):
    nd = len(shape)
    return pl.BlockSpec(shape, lambda i, _n=nd: (0,) * _n)


def _stats_of(h, stats_ref, i):
    @pl.when(i == 0)
    def _():
        stats_ref[...] = jnp.zeros_like(stats_ref)
    stats_ref[0:1, :] += jnp.sum(h, axis=0, keepdims=True)
    stats_ref[1:2, :] += jnp.sum(h * h, axis=0, keepdims=True)


def _affine(stats, n, g, b):
    m = stats[0] / n
    v = stats[1] / n - m * m
    s = g / jnp.sqrt(v + EPS)
    return s, b - m * s


def _k1(x_ref, stats_ref):
    _stats_of(x_ref[...], stats_ref, pl.program_id(0))


def _k3(ar_ref, ac_ref, w1r_ref, w1c_ref, p_ref, stats_ref):
    h1 = (jnp.dot(ar_ref[...], w1r_ref[...],
                  preferred_element_type=jnp.float32)
          + jnp.dot(ac_ref[...], w1c_ref[...],
                    preferred_element_type=jnp.float32)
          + p_ref[0:1, :])
    _stats_of(h1, stats_ref, pl.program_id(0))


def _k4(ar_ref, ac_ref, w1r_ref, w1c_ref, w2_ref, p_ref, h2_ref, stats_ref):
    h1 = (jnp.dot(ar_ref[...], w1r_ref[...],
                  preferred_element_type=jnp.float32)
          + jnp.dot(ac_ref[...], w1c_ref[...],
                    preferred_element_type=jnp.float32)
          + p_ref[0:1, :])
    e1 = jnp.maximum(h1 * p_ref[1:2, :] + p_ref[2:3, :], 0.0)
    h2 = jnp.dot(e1, w2_ref[...],
                 preferred_element_type=jnp.float32) + p_ref[3:4, :]
    h2_ref[...] = h2
    _stats_of(h2, stats_ref, pl.program_id(0))


def _k5(h2_ref, ar_ref, wx_ref, we_ref, p_ref, g1_ref, stats_ref):
    e = h2_ref[...] * p_ref[0:1, :] + p_ref[1:2, :]
    g1 = (jnp.dot(ar_ref[...], wx_ref[...],
                  preferred_element_type=jnp.float32)
          + jnp.dot(e, we_ref[...], preferred_element_type=jnp.float32)
          + p_ref[2:3, :])
    g1_ref[...] = g1
    _stats_of(g1, stats_ref, pl.program_id(0))


def _k6(g1_ref, w_ref, p_ref, g2_ref, stats_ref):
    f1 = jnp.maximum(g1_ref[...] * p_ref[0:1, :] + p_ref[1:2, :], 0.0)
    g2 = jnp.dot(f1, w_ref[...],
                 preferred_element_type=jnp.float32) + p_ref[2:3, :]
    g2_ref[...] = g2
    _stats_of(g2, stats_ref, pl.program_id(0))


def _k8(x_ref, nsum_ref, ncnt_ref, wx_ref, wa_ref, p_ref, hn1_ref, stats_ref):
    cnt = ncnt_ref[...][:, 0:1]
    mean = nsum_ref[...] / jnp.maximum(cnt, 1.0)
    agg = jnp.where(cnt > 0.0, mean * p_ref[0:1, :] + p_ref[1:2, :], 0.0)
    hn1 = (jnp.dot(x_ref[...], wx_ref[...],
                   preferred_element_type=jnp.float32)
           + jnp.dot(agg, wa_ref[...], preferred_element_type=jnp.float32)
           + p_ref[2:3, :])
    hn1_ref[...] = hn1
    _stats_of(hn1, stats_ref, pl.program_id(0))


_k9 = _k6


def _k10(hn2_ref, b_ref, p_ref, gsum_ref, gcnt_ref):
    i = pl.program_id(0)
    xn = hn2_ref[...] * p_ref[0:1, :] + p_ref[1:2, :]
    b = b_ref[0]
    onehot = (b.reshape(TN, 1)
              == lax.broadcasted_iota(jnp.int32, (TN, G), 1)
              ).astype(jnp.float32)

    @pl.when(i == 0)
    def _():
        gsum_ref[...] = jnp.zeros_like(gsum_ref)
        gcnt_ref[...] = jnp.zeros_like(gcnt_ref)

    gsum_ref[...] += lax.dot_general(onehot, xn, (((0,), (0,)), ((), ())),
                                     preferred_element_type=jnp.float32)
    gcnt_ref[...] += lax.dot_general(onehot, jnp.ones((TN, 8), jnp.float32),
                                     (((0,), (0,)), ((), ())),
                                     preferred_element_type=jnp.float32)


def _k11(gsum_ref, gcnt_ref, w1_ref, p_ref, w2_ref, b2_ref, out_ref):
    cnt = gcnt_ref[...][:, 0:1]
    u = gsum_ref[...] / jnp.maximum(cnt, 1.0)
    uu = jnp.dot(u, w1_ref[...],
                 preferred_element_type=jnp.float32) + p_ref[0:1, :]
    m = jnp.mean(uu, axis=0, keepdims=True)
    v = jnp.mean(uu * uu, axis=0, keepdims=True) - m * m
    uu = (uu - m) / jnp.sqrt(v + EPS) * p_ref[1:2, :] + p_ref[2:3, :]
    uu = jnp.maximum(uu, 0.0)
    out_ref[...] = jnp.dot(uu, w2_ref[...],
                           preferred_element_type=jnp.float32) + b2_ref[0:1, :]


def _rows(*vs):
    w = vs[0].shape[-1]
    out = jnp.zeros((8, w), jnp.float32)
    for i, v in enumerate(vs):
        out = out.at[i].set(v)
    return out


def kernel(x, params, edge_index, batch):
    p = params
    row = edge_index[0]
    col = edge_index[1]
    x_pad = jnp.pad(x, ((0, 0), (0, FP - F)))

    f32 = jnp.float32
    sds = jax.ShapeDtypeStruct

    xstats = pl.pallas_call(
        _k1, grid=(GN,),
        in_specs=[pl.BlockSpec((TN, FP), lambda i: (i, 0))],
        out_specs=_full((8, FP)),
        out_shape=sds((8, FP), f32),
    )(x_pad)
    s0, t0 = _affine(xstats[:, :F], N, p['bn0_g'], p['bn0_b'])
    s0p = jnp.pad(s0, (0, FP - F))
    t0p = jnp.pad(t0, (0, FP - F))

    W1r = jnp.pad((p['eW1'][:, :F] * s0[None, :]).T, ((0, FP - F), (0, 0)))
    W1c = jnp.pad((p['eW1'][:, F:] * s0[None, :]).T, ((0, FP - F), (0, 0)))
    b1 = p['eb1'] + (p['eW1'][:, :F] + p['eW1'][:, F:]) @ t0
    W2e = p['eW2'].T
    W1nx = jnp.pad((p['n1W1'][:, :F] * s0[None, :]).T, ((0, FP - F), (0, 0)))
    W1ne = p['n1W1'][:, F:].T
    c1 = p['n1b1'] + p['n1W1'][:, :F] @ t0
    W2n = p['n1W2'].T
    W2nx = jnp.pad((p['n2W1'][:, :F] * s0[None, :]).T, ((0, FP - F), (0, 0)))
    W2na = p['n2W1'][:, F:].T
    d1 = p['n2b1'] + p['n2W1'][:, :F] @ t0
    W2n2 = p['n2W2'].T

    ar, ac = _sc_gather(x_pad, row, col)

    eblk = pl.BlockSpec((TE, FP), lambda i: (i, 0))
    hblk = pl.BlockSpec((TE, H), lambda i: (i, 0))

    st1 = pl.pallas_call(
        _k3, grid=(GE,),
        in_specs=[eblk, eblk, _full((FP, H)), _full((FP, H)), _full((8, H))],
        out_specs=_full((8, H)),
        out_shape=sds((8, H), f32),
    )(ar, ac, W1r, W1c, _rows(b1))
    s1, t1 = _affine(st1, E, p['ebn1_g'], p['ebn1_b'])

    h2, st2 = pl.pallas_call(
        _k4, grid=(GE,),
        in_specs=[eblk, eblk, _full((FP, H)), _full((FP, H)),
                  _full((H, H)), _full((8, H))],
        out_specs=(hblk, _full((8, H))),
        out_shape=(sds((E, H), f32), sds((8, H), f32)),
    )(ar, ac, W1r, W1c, W2e, _rows(b1, s1, t1, p['eb2']))
    s2, t2 = _affine(st2, E, p['ebn2_g'], p['ebn2_b'])

    g1, st3 = pl.pallas_call(
        _k5, grid=(GE,),
        in_specs=[hblk, eblk, _full((FP, H)), _full((H, H)), _full((8, H))],
        out_specs=(hblk, _full((8, H))),
        out_shape=(sds((E, H), f32), sds((8, H), f32)),
    )(h2, ar, W1nx, W1ne, _rows(s2, t2, c1))
    u1, v1 = _affine(st3, E, p['n1bn1_g'], p['n1bn1_b'])

    g2, st4 = pl.pallas_call(
        _k6, grid=(GE,),
        in_specs=[hblk, _full((H, H)), _full((8, H))],
        out_specs=(hblk, _full((8, H))),
        out_shape=(sds((E, H), f32), sds((8, H), f32)),
    )(g1, W2n, _rows(u1, v1, p['n1b2']))
    u2, v2 = _affine(st4, E, p['n1bn2_g'], p['n1bn2_b'])

    zeros64 = jnp.zeros((64, H), f32)
    zeros16 = jnp.zeros(((RNG + RPAD) // NS, 16), f32)
    ones16 = jnp.ones((DK, 16), f32)
    nsum, ncnt16 = _sc_scatter(g2, col, zeros64, zeros16, ones16)

    nblk = pl.BlockSpec((TN, H), lambda i: (i, 0))

    hn1, st5 = pl.pallas_call(
        _k8, grid=(GN,),
        in_specs=[pl.BlockSpec((TN, FP), lambda i: (i, 0)), nblk,
                  pl.BlockSpec((TN, 16), lambda i: (i, 0)),
                  _full((FP, H)), _full((H, H)), _full((8, H))],
        out_specs=(nblk, _full((8, H))),
        out_shape=(sds((N, H), f32), sds((8, H), f32)),
    )(x_pad, nsum, ncnt16, W2nx, W2na, _rows(u2, v2, d1))
    w1a, z1a = _affine(st5, N, p['n2bn1_g'], p['n2bn1_b'])

    hn2, st6 = pl.pallas_call(
        _k9, grid=(GN,),
        in_specs=[nblk, _full((H, H)), _full((8, H))],
        out_specs=(nblk, _full((8, H))),
        out_shape=(sds((N, H), f32), sds((8, H), f32)),
    )(hn1, W2n2, _rows(w1a, z1a, p['n2b2']))
    w2a, z2a = _affine(st6, N, p['n2bn2_g'], p['n2bn2_b'])

    batch3 = batch.reshape(GN, 1, TN)
    gsum, gcnt = pl.pallas_call(
        _k10, grid=(GN,),
        in_specs=[nblk, pl.BlockSpec((1, 1, TN), lambda i: (i, 0, 0)),
                  _full((8, H))],
        out_specs=(_full((G, H)), _full((G, 8))),
        out_shape=(sds((G, H), f32), sds((G, 8), f32)),
    )(hn2, batch3, _rows(w2a, z2a))

    W2g = jnp.zeros((H, 128), f32).at[:, :OUT].set(p['gW2'].T)
    b2g = jnp.zeros((8, 128), f32).at[0, :OUT].set(p['gb2'])
    outp = pl.pallas_call(
        _k11, grid=(1,),
        in_specs=[_full((G, H)), _full((G, 8)), _full((H, H)),
                  _full((8, H)), _full((H, 128)), _full((8, 128))],
        out_specs=_full((G, 128)),
        out_shape=sds((G, 128), f32),
    )(gsum, gcnt, p['gW1'].T, _rows(p['gb1'], p['gbn1_g'], p['gbn1_b']),
      W2g, b2g)
    return outp[:, :OUT]


# speedup vs baseline: 1.7192x; 1.7105x over previous
import functools

import jax
import jax.numpy as jnp
from jax import lax
from jax.experimental import pallas as pl
from jax.experimental.pallas import tpu as pltpu, tpu_sc as plsc

N = 50000
E = 800000
F = 74
FP = 80
H = 256
G = 512
OUT = 2
EPS = 1e-5

TE = 3200
TN = 2000
GE = E // TE
GN = N // TN

NC, NS = 2, 16
NW = NC * NS
EW = E // NW
GCH = 1000
RNG = 2048
NRANGES = 26
NPAD = RNG * NRANGES
RPAD = 16
SCH = 2000
ECT = E // NS
DK = 128

_mesh = plsc.VectorSubcoreMesh(core_axis_name="c", subcore_axis_name="s")


@functools.partial(
    pl.kernel, mesh=_mesh,
    out_type=(jax.ShapeDtypeStruct((E, FP), jnp.float32),
              jax.ShapeDtypeStruct((E, FP), jnp.float32)),
    scratch_types=[pltpu.VMEM((GCH,), jnp.int32),
                   pltpu.VMEM((GCH, FP), jnp.float32),
                   pltpu.SemaphoreType.DMA],
    compiler_params=pltpu.CompilerParams(use_tc_tiling_on_sc=False),
)
def _sc_gather(xp_hbm, row_hbm, col_hbm, ar_hbm, ac_hbm, idx_v, buf, sem):
    wid = lax.axis_index("s") * NC + lax.axis_index("c")
    base = wid * EW

    def step(i, _):
        off = base + i * GCH
        pltpu.sync_copy(row_hbm.at[pl.ds(off, GCH)], idx_v)
        pltpu.async_copy(xp_hbm.at[idx_v], buf, sem).wait()
        pltpu.sync_copy(buf, ar_hbm.at[pl.ds(off, GCH)])
        pltpu.sync_copy(col_hbm.at[pl.ds(off, GCH)], idx_v)
        pltpu.async_copy(xp_hbm.at[idx_v], buf, sem).wait()
        pltpu.sync_copy(buf, ac_hbm.at[pl.ds(off, GCH)])
        return _

    lax.fori_loop(0, EW // GCH, step, 0)


@functools.partial(
    pl.kernel, mesh=_mesh,
    out_type=(jax.ShapeDtypeStruct((NPAD, H), jnp.float32),
              jax.ShapeDtypeStruct((NPAD, 16), jnp.float32)),
    scratch_types=[pltpu.VMEM((SCH,), jnp.int32),
                   pltpu.VMEM((SCH + 160,), jnp.int32),
                   pltpu.VMEM((SCH + 160,), jnp.int32),
                   pltpu.VMEM((DK,), jnp.int32),
                   pltpu.VMEM((DK,), jnp.int32),
                   pltpu.VMEM((DK, H), jnp.float32),
                   pltpu.VMEM((DK, H), jnp.float32),
                   pltpu.VMEM((16, H), jnp.float32),
                   pltpu.VMEM(((RNG + RPAD) // NS, 16), jnp.float32),
                   pltpu.VMEM((DK, 16), jnp.float32),
                   pltpu.VMEM_SHARED((RNG + RPAD, H), jnp.float32),
                   pltpu.VMEM_SHARED((RNG + RPAD, 16), jnp.float32),
                   pltpu.SemaphoreType.DMA,
                   pltpu.SemaphoreType.DMA],
    compiler_params=pltpu.CompilerParams(use_tc_tiling_on_sc=False,
                                         needs_layout_passes=False),
)
def _sc_scatter(g2_hbm, col_hbm, z_hbm, z16_hbm, on_hbm,
                nsum_hbm, ncnt_hbm,
                cbuf, midx, mloc, loc_a, loc_b, dbuf_a, dbuf_b,
                zbuf, z16v, ones_v, acc, cnta, sem_a, sem_b):
    c = lax.axis_index("c")
    s = lax.axis_index("s")
    iota = lax.iota(jnp.int32, 16)
    tbase = s * ECT
    pltpu.sync_copy(z_hbm, zbuf)
    pltpu.sync_copy(z16_hbm, z16v)
    pltpu.sync_copy(on_hbm, ones_v)

    def _drain_one(k0, loc, dbuf, sem):
        for j in range(DK // 16):
            loc[pl.ds(j * 16, 16)] = mloc[pl.ds(k0 * DK + j * 16, 16)]
        return pltpu.async_copy(g2_hbm.at[midx.at[pl.ds(k0 * DK, DK)]],
                                dbuf, sem)

    def _add_one(loc, dbuf):
        pltpu.sync_copy(dbuf, acc.at[loc], add=True)
        pltpu.sync_copy(ones_v, cnta.at[loc], add=True)

    def range_step(r_i, carry_r):
        rng = 2 * r_i + c
        lo = rng * RNG
        def zstep(k, _z):
            pltpu.sync_copy(zbuf,
                            acc.at[pl.ds(s * (RNG // NS) + k * 16, 16)])
            return _z
        lax.fori_loop(0, RNG // NS // 16, zstep, 0)
        pltpu.sync_copy(z16v, cnta.at[pl.ds(s * ((RNG + RPAD) // NS),
                                            (RNG + RPAD) // NS)])
        plsc.subcore_barrier()

        def chunk_step(ch, cur_in):
            ebase = tbase + ch * SCH
            pltpu.sync_copy(col_hbm.at[pl.ds(ebase, SCH)], cbuf)

            def scan_step(v, cur):
                cols = cbuf[pl.ds(v * 16, 16)]
                m = (cols >= lo) & (cols < lo + RNG)
                pre = plsc.cumsum(m.astype(jnp.int32))
                eid = ebase + v * 16 + iota
                dst = jnp.where(m, cur + pre - 1, SCH + 128 + iota)
                plsc.store_scatter(midx, [dst], eid)
                plsc.store_scatter(mloc, [dst], cols - lo)
                return cur + jnp.max(pre)

            cur = lax.fori_loop(0, SCH // 16, scan_step, cur_in)
            n128 = cur // DK

            def pair_step(p, _d):
                cp_a = _drain_one(2 * p, loc_a, dbuf_a, sem_a)
                cp_b = _drain_one(2 * p + 1, loc_b, dbuf_b, sem_b)
                cp_a.wait()
                _add_one(loc_a, dbuf_a)
                cp_b.wait()
                _add_one(loc_b, dbuf_b)
                return _d

            lax.fori_loop(0, n128 // 2, pair_step, 0)

            @pl.when(n128 % 2 == 1)
            def _():
                _drain_one(n128 - 1, loc_a, dbuf_a, sem_a).wait()
                _add_one(loc_a, dbuf_a)

            rem = cur - n128 * DK
            for j in range(DK // 16):
                tmp_i = midx[pl.ds(n128 * DK + j * 16, 16)]
                tmp_l = mloc[pl.ds(n128 * DK + j * 16, 16)]
                midx[pl.ds(j * 16, 16)] = tmp_i
                mloc[pl.ds(j * 16, 16)] = tmp_l
            return rem

        rem = lax.fori_loop(0, ECT // SCH, chunk_step, 0)

        for j in range(DK // 16):
            midx[pl.ds(rem + j * 16, 16)] = jnp.zeros((16,), jnp.int32)
            mloc[pl.ds(rem + j * 16, 16)] = jnp.full((16,), RNG, jnp.int32)

        @pl.when(rem > 0)
        def _():
            _drain_one(0, loc_a, dbuf_a, sem_a).wait()
            _add_one(loc_a, dbuf_a)

        plsc.subcore_barrier()

        out0 = rng * RNG + s * (RNG // NS)
        pltpu.sync_copy(acc.at[pl.ds(s * (RNG // NS), RNG // NS)],
                        nsum_hbm.at[pl.ds(out0, RNG // NS)])
        pltpu.sync_copy(cnta.at[pl.ds(s * (RNG // NS), RNG // NS)],
                        ncnt_hbm.at[pl.ds(out0, RNG // NS)])
        plsc.subcore_barrier()
        return carry_r

    lax.fori_loop(0, NRANGES // NC, range_step, 0)


def _full(shape):
    nd = len(shape)
    return pl.BlockSpec(shape, lambda i, _n=nd: (0,) * _n)


def _stats_of(h, stats_ref, i):
    @pl.when(i == 0)
    def _():
        stats_ref[...] = jnp.zeros_like(stats_ref)
    stats_ref[0:1, :] += jnp.sum(h, axis=0, keepdims=True)
    stats_ref[1:2, :] += jnp.sum(h * h, axis=0, keepdims=True)


def _affine(stats, n, g, b):
    m = stats[0] / n
    v = stats[1] / n - m * m
    s = g / jnp.sqrt(v + EPS)
    return s, b - m * s


def _k1(x_ref, stats_ref):
    _stats_of(x_ref[...], stats_ref, pl.program_id(0))


def _k3(ar_ref, ac_ref, w1r_ref, w1c_ref, p_ref, stats_ref):
    h1 = (jnp.dot(ar_ref[...], w1r_ref[...],
                  preferred_element_type=jnp.float32)
          + jnp.dot(ac_ref[...], w1c_ref[...],
                    preferred_element_type=jnp.float32)
          + p_ref[0:1, :])
    _stats_of(h1, stats_ref, pl.program_id(0))


def _k4(ar_ref, ac_ref, w1r_ref, w1c_ref, w2_ref, p_ref, h2_ref, stats_ref):
    h1 = (jnp.dot(ar_ref[...], w1r_ref[...],
                  preferred_element_type=jnp.float32)
          + jnp.dot(ac_ref[...], w1c_ref[...],
                    preferred_element_type=jnp.float32)
          + p_ref[0:1, :])
    e1 = jnp.maximum(h1 * p_ref[1:2, :] + p_ref[2:3, :], 0.0)
    h2 = jnp.dot(e1, w2_ref[...],
                 preferred_element_type=jnp.float32) + p_ref[3:4, :]
    h2_ref[...] = h2
    _stats_of(h2, stats_ref, pl.program_id(0))


def _k5(h2_ref, ar_ref, wx_ref, we_ref, p_ref, g1_ref, stats_ref):
    e = h2_ref[...] * p_ref[0:1, :] + p_ref[1:2, :]
    g1 = (jnp.dot(ar_ref[...], wx_ref[...],
                  preferred_element_type=jnp.float32)
          + jnp.dot(e, we_ref[...], preferred_element_type=jnp.float32)
          + p_ref[2:3, :])
    g1_ref[...] = g1
    _stats_of(g1, stats_ref, pl.program_id(0))


def _k6(g1_ref, w_ref, p_ref, g2_ref, stats_ref):
    f1 = jnp.maximum(g1_ref[...] * p_ref[0:1, :] + p_ref[1:2, :], 0.0)
    g2 = jnp.dot(f1, w_ref[...],
                 preferred_element_type=jnp.float32) + p_ref[2:3, :]
    g2_ref[...] = g2
    _stats_of(g2, stats_ref, pl.program_id(0))


def _k8(x_ref, nsum_ref, ncnt_ref, wx_ref, wa_ref, p_ref, hn1_ref, stats_ref):
    cnt = ncnt_ref[...][:, 0:1]
    mean = nsum_ref[...] / jnp.maximum(cnt, 1.0)
    agg = jnp.where(cnt > 0.0, mean * p_ref[0:1, :] + p_ref[1:2, :], 0.0)
    hn1 = (jnp.dot(x_ref[...], wx_ref[...],
                   preferred_element_type=jnp.float32)
           + jnp.dot(agg, wa_ref[...], preferred_element_type=jnp.float32)
           + p_ref[2:3, :])
    hn1_ref[...] = hn1
    _stats_of(hn1, stats_ref, pl.program_id(0))


_k9 = _k6


def _k10(hn2_ref, b_ref, p_ref, gsum_ref, gcnt_ref):
    i = pl.program_id(0)
    xn = hn2_ref[...] * p_ref[0:1, :] + p_ref[1:2, :]
    b = b_ref[0]
    onehot = (b.reshape(TN, 1)
              == lax.broadcasted_iota(jnp.int32, (TN, G), 1)
              ).astype(jnp.float32)

    @pl.when(i == 0)
    def _():
        gsum_ref[...] = jnp.zeros_like(gsum_ref)
        gcnt_ref[...] = jnp.zeros_like(gcnt_ref)

    gsum_ref[...] += lax.dot_general(onehot, xn, (((0,), (0,)), ((), ())),
                                     preferred_element_type=jnp.float32)
    gcnt_ref[...] += lax.dot_general(onehot, jnp.ones((TN, 8), jnp.float32),
                                     (((0,), (0,)), ((), ())),
                                     preferred_element_type=jnp.float32)


def _k11(gsum_ref, gcnt_ref, w1_ref, p_ref, w2_ref, b2_ref, out_ref):
    cnt = gcnt_ref[...][:, 0:1]
    u = gsum_ref[...] / jnp.maximum(cnt, 1.0)
    uu = jnp.dot(u, w1_ref[...],
                 preferred_element_type=jnp.float32) + p_ref[0:1, :]
    m = jnp.mean(uu, axis=0, keepdims=True)
    v = jnp.mean(uu * uu, axis=0, keepdims=True) - m * m
    uu = (uu - m) / jnp.sqrt(v + EPS) * p_ref[1:2, :] + p_ref[2:3, :]
    uu = jnp.maximum(uu, 0.0)
    out_ref[...] = jnp.dot(uu, w2_ref[...],
                           preferred_element_type=jnp.float32) + b2_ref[0:1, :]


def _rows(*vs):
    w = vs[0].shape[-1]
    out = jnp.zeros((8, w), jnp.float32)
    for i, v in enumerate(vs):
        out = out.at[i].set(v)
    return out


def kernel(x, params, edge_index, batch):
    p = params
    row = edge_index[0]
    col = edge_index[1]
    x_pad = jnp.pad(x, ((0, 0), (0, FP - F)))

    f32 = jnp.float32
    sds = jax.ShapeDtypeStruct

    xstats = pl.pallas_call(
        _k1, grid=(GN,),
        in_specs=[pl.BlockSpec((TN, FP), lambda i: (i, 0))],
        out_specs=_full((8, FP)),
        out_shape=sds((8, FP), f32),
    )(x_pad)
    s0, t0 = _affine(xstats[:, :F], N, p['bn0_g'], p['bn0_b'])
    s0p = jnp.pad(s0, (0, FP - F))
    t0p = jnp.pad(t0, (0, FP - F))

    W1r = jnp.pad((p['eW1'][:, :F] * s0[None, :]).T, ((0, FP - F), (0, 0)))
    W1c = jnp.pad((p['eW1'][:, F:] * s0[None, :]).T, ((0, FP - F), (0, 0)))
    b1 = p['eb1'] + (p['eW1'][:, :F] + p['eW1'][:, F:]) @ t0
    W2e = p['eW2'].T
    W1nx = jnp.pad((p['n1W1'][:, :F] * s0[None, :]).T, ((0, FP - F), (0, 0)))
    W1ne = p['n1W1'][:, F:].T
    c1 = p['n1b1'] + p['n1W1'][:, :F] @ t0
    W2n = p['n1W2'].T
    W2nx = jnp.pad((p['n2W1'][:, :F] * s0[None, :]).T, ((0, FP - F), (0, 0)))
    W2na = p['n2W1'][:, F:].T
    d1 = p['n2b1'] + p['n2W1'][:, :F] @ t0
    W2n2 = p['n2W2'].T

    ar, ac = _sc_gather(x_pad, row, col)

    eblk = pl.BlockSpec((TE, FP), lambda i: (i, 0))
    hblk = pl.BlockSpec((TE, H), lambda i: (i, 0))

    st1 = pl.pallas_call(
        _k3, grid=(GE,),
        in_specs=[eblk, eblk, _full((FP, H)), _full((FP, H)), _full((8, H))],
        out_specs=_full((8, H)),
        out_shape=sds((8, H), f32),
    )(ar, ac, W1r, W1c, _rows(b1))
    s1, t1 = _affine(st1, E, p['ebn1_g'], p['ebn1_b'])

    h2, st2 = pl.pallas_call(
        _k4, grid=(GE,),
        in_specs=[eblk, eblk, _full((FP, H)), _full((FP, H)),
                  _full((H, H)), _full((8, H))],
        out_specs=(hblk, _full((8, H))),
        out_shape=(sds((E, H), f32), sds((8, H), f32)),
    )(ar, ac, W1r, W1c, W2e, _rows(b1, s1, t1, p['eb2']))
    s2, t2 = _affine(st2, E, p['ebn2_g'], p['ebn2_b'])

    g1, st3 = pl.pallas_call(
        _k5, grid=(GE,),
        in_specs=[hblk, eblk, _full((FP, H)), _full((H, H)), _full((8, H))],
        out_specs=(hblk, _full((8, H))),
        out_shape=(sds((E, H), f32), sds((8, H), f32)),
    )(h2, ar, W1nx, W1ne, _rows(s2, t2, c1))
    u1, v1 = _affine(st3, E, p['n1bn1_g'], p['n1bn1_b'])

    g2, st4 = pl.pallas_call(
        _k6, grid=(GE,),
        in_specs=[hblk, _full((H, H)), _full((8, H))],
        out_specs=(hblk, _full((8, H))),
        out_shape=(sds((E, H), f32), sds((8, H), f32)),
    )(g1, W2n, _rows(u1, v1, p['n1b2']))
    u2, v2 = _affine(st4, E, p['n1bn2_g'], p['n1bn2_b'])

    zeros64 = jnp.zeros((16, H), f32)
    zeros16 = jnp.zeros(((RNG + RPAD) // NS, 16), f32)
    ones16 = jnp.ones((DK, 16), f32)
    nsum, ncnt16 = _sc_scatter(g2, col, zeros64, zeros16, ones16)

    nblk = pl.BlockSpec((TN, H), lambda i: (i, 0))

    hn1, st5 = pl.pallas_call(
        _k8, grid=(GN,),
        in_specs=[pl.BlockSpec((TN, FP), lambda i: (i, 0)), nblk,
                  pl.BlockSpec((TN, 16), lambda i: (i, 0)),
                  _full((FP, H)), _full((H, H)), _full((8, H))],
        out_specs=(nblk, _full((8, H))),
        out_shape=(sds((N, H), f32), sds((8, H), f32)),
    )(x_pad, nsum, ncnt16, W2nx, W2na, _rows(u2, v2, d1))
    w1a, z1a = _affine(st5, N, p['n2bn1_g'], p['n2bn1_b'])

    hn2, st6 = pl.pallas_call(
        _k9, grid=(GN,),
        in_specs=[nblk, _full((H, H)), _full((8, H))],
        out_specs=(nblk, _full((8, H))),
        out_shape=(sds((N, H), f32), sds((8, H), f32)),
    )(hn1, W2n2, _rows(w1a, z1a, p['n2b2']))
    w2a, z2a = _affine(st6, N, p['n2bn2_g'], p['n2bn2_b'])

    batch3 = batch.reshape(GN, 1, TN)
    gsum, gcnt = pl.pallas_call(
        _k10, grid=(GN,),
        in_specs=[nblk, pl.BlockSpec((1, 1, TN), lambda i: (i, 0, 0)),
                  _full((8, H))],
        out_specs=(_full((G, H)), _full((G, 8))),
        out_shape=(sds((G, H), f32), sds((G, 8), f32)),
    )(hn2, batch3, _rows(w2a, z2a))

    W2g = jnp.zeros((H, 128), f32).at[:, :OUT].set(p['gW2'].T)
    b2g = jnp.zeros((8, 128), f32).at[0, :OUT].set(p['gb2'])
    outp = pl.pallas_call(
        _k11, grid=(1,),
        in_specs=[_full((G, H)), _full((G, 8)), _full((H, H)),
                  _full((8, H)), _full((H, 128)), _full((8, 128))],
        out_specs=_full((G, 128)),
        out_shape=sds((G, 128), f32),
    )(gsum, gcnt, p['gW1'].T, _rows(p['gb1'], p['gbn1_g'], p['gbn1_b']),
      W2g, b2g)
    return outp[:, :OUT]


# speedup vs baseline: 1.7990x; 1.0464x over previous
import functools

import jax
import jax.numpy as jnp
from jax import lax
from jax.experimental import pallas as pl
from jax.experimental.pallas import tpu as pltpu, tpu_sc as plsc

N = 50000
E = 800000
F = 74
FP = 128
H = 256
G = 512
OUT = 2
EPS = 1e-5

TE = 3200
TN = 2000
GE = E // TE
GN = N // TN

NC, NS = 2, 16
NW = NC * NS
EW = E // NW
GCH = 128
RNG = 2048
NRANGES = 26
NPAD = RNG * NRANGES
RPAD = 16
SCH = 2000
ECT = E // NS
DK = 128

_mesh = plsc.VectorSubcoreMesh(core_axis_name="c", subcore_axis_name="s")


@functools.partial(
    pl.kernel, mesh=_mesh,
    out_type=(jax.ShapeDtypeStruct((E, FP), jnp.float32),
              jax.ShapeDtypeStruct((E, FP), jnp.float32)),
    scratch_types=[pltpu.VMEM((GCH,), jnp.int32),
                   pltpu.VMEM((GCH,), jnp.int32),
                   pltpu.VMEM((GCH, FP), jnp.float32),
                   pltpu.VMEM((GCH, FP), jnp.float32),
                   pltpu.SemaphoreType.DMA,
                   pltpu.SemaphoreType.DMA],
)
def _sc_gather(xp_hbm, row_hbm, col_hbm, ar_hbm, ac_hbm,
               idx_a, idx_b, buf_a, buf_b, sem_a, sem_b):
    wid = lax.axis_index("s") * NC + lax.axis_index("c")
    base = wid * EW
    NFULL = EW // GCH
    TAIL = EW - NFULL * GCH

    def step(i, _):
        off = base + i * GCH
        pltpu.sync_copy(row_hbm.at[pl.ds(off, GCH)], idx_a)
        pltpu.sync_copy(col_hbm.at[pl.ds(off, GCH)], idx_b)
        cp_a = pltpu.async_copy(xp_hbm.at[idx_a], buf_a, sem_a)
        cp_b = pltpu.async_copy(xp_hbm.at[idx_b], buf_b, sem_b)
        cp_a.wait()
        pltpu.sync_copy(buf_a, ar_hbm.at[pl.ds(off, GCH)])
        cp_b.wait()
        pltpu.sync_copy(buf_b, ac_hbm.at[pl.ds(off, GCH)])
        return _

    lax.fori_loop(0, NFULL, step, 0)

    if TAIL:
        off = base + NFULL * GCH
        pltpu.sync_copy(row_hbm.at[pl.ds(off, TAIL)],
                        idx_a.at[pl.ds(0, TAIL)])
        pltpu.sync_copy(col_hbm.at[pl.ds(off, TAIL)],
                        idx_b.at[pl.ds(0, TAIL)])
        cp_a = pltpu.async_copy(xp_hbm.at[idx_a.at[pl.ds(0, TAIL)]],
                                buf_a.at[pl.ds(0, TAIL)], sem_a)
        cp_b = pltpu.async_copy(xp_hbm.at[idx_b.at[pl.ds(0, TAIL)]],
                                buf_b.at[pl.ds(0, TAIL)], sem_b)
        cp_a.wait()
        pltpu.sync_copy(buf_a.at[pl.ds(0, TAIL)],
                        ar_hbm.at[pl.ds(off, TAIL)])
        cp_b.wait()
        pltpu.sync_copy(buf_b.at[pl.ds(0, TAIL)],
                        ac_hbm.at[pl.ds(off, TAIL)])


@functools.partial(
    pl.kernel, mesh=_mesh,
    out_type=(jax.ShapeDtypeStruct((NPAD, H), jnp.float32),
              jax.ShapeDtypeStruct((NPAD, 16), jnp.float32)),
    scratch_types=[pltpu.VMEM((SCH,), jnp.int32),
                   pltpu.VMEM((SCH + 160,), jnp.int32),
                   pltpu.VMEM((SCH + 160,), jnp.int32),
                   pltpu.VMEM((DK,), jnp.int32),
                   pltpu.VMEM((DK,), jnp.int32),
                   pltpu.VMEM((DK, H), jnp.float32),
                   pltpu.VMEM((DK, H), jnp.float32),
                   pltpu.VMEM((16, H), jnp.float32),
                   pltpu.VMEM(((RNG + RPAD) // NS, 16), jnp.float32),
                   pltpu.VMEM((DK, 16), jnp.float32),
                   pltpu.VMEM_SHARED((RNG + RPAD, H), jnp.float32),
                   pltpu.VMEM_SHARED((RNG + RPAD, 16), jnp.float32),
                   pltpu.SemaphoreType.DMA,
                   pltpu.SemaphoreType.DMA],
    compiler_params=pltpu.CompilerParams(use_tc_tiling_on_sc=False,
                                         needs_layout_passes=False),
)
def _sc_scatter(g2_hbm, col_hbm, z_hbm, z16_hbm, on_hbm,
                nsum_hbm, ncnt_hbm,
                cbuf, midx, mloc, loc_a, loc_b, dbuf_a, dbuf_b,
                zbuf, z16v, ones_v, acc, cnta, sem_a, sem_b):
    c = lax.axis_index("c")
    s = lax.axis_index("s")
    iota = lax.iota(jnp.int32, 16)
    tbase = s * ECT
    pltpu.sync_copy(z_hbm, zbuf)
    pltpu.sync_copy(z16_hbm, z16v)
    pltpu.sync_copy(on_hbm, ones_v)

    def _drain_one(k0, loc, dbuf, sem):
        for j in range(DK // 16):
            loc[pl.ds(j * 16, 16)] = mloc[pl.ds(k0 * DK + j * 16, 16)]
        return pltpu.async_copy(g2_hbm.at[midx.at[pl.ds(k0 * DK, DK)]],
                                dbuf, sem)

    def _add_one(loc, dbuf):
        pltpu.sync_copy(dbuf, acc.at[loc], add=True)
        pltpu.sync_copy(ones_v, cnta.at[loc], add=True)

    def range_step(r_i, carry_r):
        rng = 2 * r_i + c
        lo = rng * RNG
        def zstep(k, _z):
            pltpu.sync_copy(zbuf,
                            acc.at[pl.ds(s * (RNG // NS) + k * 16, 16)])
            return _z
        lax.fori_loop(0, RNG // NS // 16, zstep, 0)
        pltpu.sync_copy(z16v, cnta.at[pl.ds(s * ((RNG + RPAD) // NS),
                                            (RNG + RPAD) // NS)])
        plsc.subcore_barrier()

        def chunk_step(ch, cur_in):
            ebase = tbase + ch * SCH
            pltpu.sync_copy(col_hbm.at[pl.ds(ebase, SCH)], cbuf)

            def scan_step(v, cur):
                cols = cbuf[pl.ds(v * 16, 16)]
                m = (cols >= lo) & (cols < lo + RNG)
                pre = plsc.cumsum(m.astype(jnp.int32))
                eid = ebase + v * 16 + iota
                dst = jnp.where(m, cur + pre - 1, SCH + 128 + iota)
                plsc.store_scatter(midx, [dst], eid)
                plsc.store_scatter(mloc, [dst], cols - lo)
                return cur + jnp.max(pre)

            cur = lax.fori_loop(0, SCH // 16, scan_step, cur_in)
            n128 = cur // DK

            def pair_step(p, _d):
                cp_a = _drain_one(2 * p, loc_a, dbuf_a, sem_a)
                cp_b = _drain_one(2 * p + 1, loc_b, dbuf_b, sem_b)
                cp_a.wait()
                _add_one(loc_a, dbuf_a)
                cp_b.wait()
                _add_one(loc_b, dbuf_b)
                return _d

            lax.fori_loop(0, n128 // 2, pair_step, 0)

            @pl.when(n128 % 2 == 1)
            def _():
                _drain_one(n128 - 1, loc_a, dbuf_a, sem_a).wait()
                _add_one(loc_a, dbuf_a)

            rem = cur - n128 * DK
            for j in range(DK // 16):
                tmp_i = midx[pl.ds(n128 * DK + j * 16, 16)]
                tmp_l = mloc[pl.ds(n128 * DK + j * 16, 16)]
                midx[pl.ds(j * 16, 16)] = tmp_i
                mloc[pl.ds(j * 16, 16)] = tmp_l
            return rem

        rem = lax.fori_loop(0, ECT // SCH, chunk_step, 0)

        for j in range(DK // 16):
            midx[pl.ds(rem + j * 16, 16)] = jnp.zeros((16,), jnp.int32)
            mloc[pl.ds(rem + j * 16, 16)] = jnp.full((16,), RNG, jnp.int32)

        @pl.when(rem > 0)
        def _():
            _drain_one(0, loc_a, dbuf_a, sem_a).wait()
            _add_one(loc_a, dbuf_a)

        plsc.subcore_barrier()

        out0 = rng * RNG + s * (RNG // NS)
        pltpu.sync_copy(acc.at[pl.ds(s * (RNG // NS), RNG // NS)],
                        nsum_hbm.at[pl.ds(out0, RNG // NS)])
        pltpu.sync_copy(cnta.at[pl.ds(s * (RNG // NS), RNG // NS)],
                        ncnt_hbm.at[pl.ds(out0, RNG // NS)])
        plsc.subcore_barrier()
        return carry_r

    lax.fori_loop(0, NRANGES // NC, range_step, 0)


def _full(shape):
    nd = len(shape)
    return pl.BlockSpec(shape, lambda i, _n=nd: (0,) * _n)


def _stats_of(h, stats_ref, i):
    @pl.when(i == 0)
    def _():
        stats_ref[...] = jnp.zeros_like(stats_ref)
    stats_ref[0:1, :] += jnp.sum(h, axis=0, keepdims=True)
    stats_ref[1:2, :] += jnp.sum(h * h, axis=0, keepdims=True)


def _affine(stats, n, g, b):
    m = stats[0] / n
    v = stats[1] / n - m * m
    s = g / jnp.sqrt(v + EPS)
    return s, b - m * s


def _k1(x_ref, stats_ref):
    _stats_of(x_ref[...], stats_ref, pl.program_id(0))


def _k3(ar_ref, ac_ref, w1r_ref, w1c_ref, p_ref, stats_ref):
    h1 = (jnp.dot(ar_ref[...], w1r_ref[...],
                  preferred_element_type=jnp.float32)
          + jnp.dot(ac_ref[...], w1c_ref[...],
                    preferred_element_type=jnp.float32)
          + p_ref[0:1, :])
    _stats_of(h1, stats_ref, pl.program_id(0))


def _k4(ar_ref, ac_ref, w1r_ref, w1c_ref, w2_ref, p_ref, h2_ref, stats_ref):
    h1 = (jnp.dot(ar_ref[...], w1r_ref[...],
                  preferred_element_type=jnp.float32)
          + jnp.dot(ac_ref[...], w1c_ref[...],
                    preferred_element_type=jnp.float32)
          + p_ref[0:1, :])
    e1 = jnp.maximum(h1 * p_ref[1:2, :] + p_ref[2:3, :], 0.0)
    h2 = jnp.dot(e1, w2_ref[...],
                 preferred_element_type=jnp.float32) + p_ref[3:4, :]
    h2_ref[...] = h2
    _stats_of(h2, stats_ref, pl.program_id(0))


def _k5(h2_ref, ar_ref, wx_ref, we_ref, p_ref, g1_ref, stats_ref):
    e = h2_ref[...] * p_ref[0:1, :] + p_ref[1:2, :]
    g1 = (jnp.dot(ar_ref[...], wx_ref[...],
                  preferred_element_type=jnp.float32)
          + jnp.dot(e, we_ref[...], preferred_element_type=jnp.float32)
          + p_ref[2:3, :])
    g1_ref[...] = g1
    _stats_of(g1, stats_ref, pl.program_id(0))


def _k6(g1_ref, w_ref, p_ref, g2_ref, stats_ref):
    f1 = jnp.maximum(g1_ref[...] * p_ref[0:1, :] + p_ref[1:2, :], 0.0)
    g2 = jnp.dot(f1, w_ref[...],
                 preferred_element_type=jnp.float32) + p_ref[2:3, :]
    g2_ref[...] = g2
    _stats_of(g2, stats_ref, pl.program_id(0))


def _k8(x_ref, nsum_ref, ncnt_ref, wx_ref, wa_ref, p_ref, hn1_ref, stats_ref):
    cnt = ncnt_ref[...][:, 0:1]
    mean = nsum_ref[...] / jnp.maximum(cnt, 1.0)
    agg = jnp.where(cnt > 0.0, mean * p_ref[0:1, :] + p_ref[1:2, :], 0.0)
    hn1 = (jnp.dot(x_ref[...], wx_ref[...],
                   preferred_element_type=jnp.float32)
           + jnp.dot(agg, wa_ref[...], preferred_element_type=jnp.float32)
           + p_ref[2:3, :])
    hn1_ref[...] = hn1
    _stats_of(hn1, stats_ref, pl.program_id(0))


_k9 = _k6


def _k10(hn2_ref, b_ref, p_ref, gsum_ref, gcnt_ref):
    i = pl.program_id(0)
    xn = hn2_ref[...] * p_ref[0:1, :] + p_ref[1:2, :]
    b = b_ref[0]
    onehot = (b.reshape(TN, 1)
              == lax.broadcasted_iota(jnp.int32, (TN, G), 1)
              ).astype(jnp.float32)

    @pl.when(i == 0)
    def _():
        gsum_ref[...] = jnp.zeros_like(gsum_ref)
        gcnt_ref[...] = jnp.zeros_like(gcnt_ref)

    gsum_ref[...] += lax.dot_general(onehot, xn, (((0,), (0,)), ((), ())),
                                     preferred_element_type=jnp.float32)
    gcnt_ref[...] += lax.dot_general(onehot, jnp.ones((TN, 8), jnp.float32),
                                     (((0,), (0,)), ((), ())),
                                     preferred_element_type=jnp.float32)


def _k11(gsum_ref, gcnt_ref, w1_ref, p_ref, w2_ref, b2_ref, out_ref):
    cnt = gcnt_ref[...][:, 0:1]
    u = gsum_ref[...] / jnp.maximum(cnt, 1.0)
    uu = jnp.dot(u, w1_ref[...],
                 preferred_element_type=jnp.float32) + p_ref[0:1, :]
    m = jnp.mean(uu, axis=0, keepdims=True)
    v = jnp.mean(uu * uu, axis=0, keepdims=True) - m * m
    uu = (uu - m) / jnp.sqrt(v + EPS) * p_ref[1:2, :] + p_ref[2:3, :]
    uu = jnp.maximum(uu, 0.0)
    out_ref[...] = jnp.dot(uu, w2_ref[...],
                           preferred_element_type=jnp.float32) + b2_ref[0:1, :]


def _rows(*vs):
    w = vs[0].shape[-1]
    out = jnp.zeros((8, w), jnp.float32)
    for i, v in enumerate(vs):
        out = out.at[i].set(v)
    return out


def kernel(x, params, edge_index, batch):
    p = params
    row = edge_index[0]
    col = edge_index[1]
    x_pad = jnp.pad(x, ((0, 0), (0, FP - F)))

    f32 = jnp.float32
    sds = jax.ShapeDtypeStruct

    xstats = pl.pallas_call(
        _k1, grid=(GN,),
        in_specs=[pl.BlockSpec((TN, FP), lambda i: (i, 0))],
        out_specs=_full((8, FP)),
        out_shape=sds((8, FP), f32),
    )(x_pad)
    s0, t0 = _affine(xstats[:, :F], N, p['bn0_g'], p['bn0_b'])
    s0p = jnp.pad(s0, (0, FP - F))
    t0p = jnp.pad(t0, (0, FP - F))

    W1r = jnp.pad((p['eW1'][:, :F] * s0[None, :]).T, ((0, FP - F), (0, 0)))
    W1c = jnp.pad((p['eW1'][:, F:] * s0[None, :]).T, ((0, FP - F), (0, 0)))
    b1 = p['eb1'] + (p['eW1'][:, :F] + p['eW1'][:, F:]) @ t0
    W2e = p['eW2'].T
    W1nx = jnp.pad((p['n1W1'][:, :F] * s0[None, :]).T, ((0, FP - F), (0, 0)))
    W1ne = p['n1W1'][:, F:].T
    c1 = p['n1b1'] + p['n1W1'][:, :F] @ t0
    W2n = p['n1W2'].T
    W2nx = jnp.pad((p['n2W1'][:, :F] * s0[None, :]).T, ((0, FP - F), (0, 0)))
    W2na = p['n2W1'][:, F:].T
    d1 = p['n2b1'] + p['n2W1'][:, :F] @ t0
    W2n2 = p['n2W2'].T

    ar, ac = _sc_gather(x_pad, row, col)

    eblk = pl.BlockSpec((TE, FP), lambda i: (i, 0))
    hblk = pl.BlockSpec((TE, H), lambda i: (i, 0))

    st1 = pl.pallas_call(
        _k3, grid=(GE,),
        in_specs=[eblk, eblk, _full((FP, H)), _full((FP, H)), _full((8, H))],
        out_specs=_full((8, H)),
        out_shape=sds((8, H), f32),
    )(ar, ac, W1r, W1c, _rows(b1))
    s1, t1 = _affine(st1, E, p['ebn1_g'], p['ebn1_b'])

    h2, st2 = pl.pallas_call(
        _k4, grid=(GE,),
        in_specs=[eblk, eblk, _full((FP, H)), _full((FP, H)),
                  _full((H, H)), _full((8, H))],
        out_specs=(hblk, _full((8, H))),
        out_shape=(sds((E, H), f32), sds((8, H), f32)),
    )(ar, ac, W1r, W1c, W2e, _rows(b1, s1, t1, p['eb2']))
    s2, t2 = _affine(st2, E, p['ebn2_g'], p['ebn2_b'])

    g1, st3 = pl.pallas_call(
        _k5, grid=(GE,),
        in_specs=[hblk, eblk, _full((FP, H)), _full((H, H)), _full((8, H))],
        out_specs=(hblk, _full((8, H))),
        out_shape=(sds((E, H), f32), sds((8, H), f32)),
    )(h2, ar, W1nx, W1ne, _rows(s2, t2, c1))
    u1, v1 = _affine(st3, E, p['n1bn1_g'], p['n1bn1_b'])

    g2, st4 = pl.pallas_call(
        _k6, grid=(GE,),
        in_specs=[hblk, _full((H, H)), _full((8, H))],
        out_specs=(hblk, _full((8, H))),
        out_shape=(sds((E, H), f32), sds((8, H), f32)),
    )(g1, W2n, _rows(u1, v1, p['n1b2']))
    u2, v2 = _affine(st4, E, p['n1bn2_g'], p['n1bn2_b'])

    zeros64 = jnp.zeros((16, H), f32)
    zeros16 = jnp.zeros(((RNG + RPAD) // NS, 16), f32)
    ones16 = jnp.ones((DK, 16), f32)
    nsum, ncnt16 = _sc_scatter(g2, col, zeros64, zeros16, ones16)

    nblk = pl.BlockSpec((TN, H), lambda i: (i, 0))

    hn1, st5 = pl.pallas_call(
        _k8, grid=(GN,),
        in_specs=[pl.BlockSpec((TN, FP), lambda i: (i, 0)), nblk,
                  pl.BlockSpec((TN, 16), lambda i: (i, 0)),
                  _full((FP, H)), _full((H, H)), _full((8, H))],
        out_specs=(nblk, _full((8, H))),
        out_shape=(sds((N, H), f32), sds((8, H), f32)),
    )(x_pad, nsum, ncnt16, W2nx, W2na, _rows(u2, v2, d1))
    w1a, z1a = _affine(st5, N, p['n2bn1_g'], p['n2bn1_b'])

    hn2, st6 = pl.pallas_call(
        _k9, grid=(GN,),
        in_specs=[nblk, _full((H, H)), _full((8, H))],
        out_specs=(nblk, _full((8, H))),
        out_shape=(sds((N, H), f32), sds((8, H), f32)),
    )(hn1, W2n2, _rows(w1a, z1a, p['n2b2']))
    w2a, z2a = _affine(st6, N, p['n2bn2_g'], p['n2bn2_b'])

    batch3 = batch.reshape(GN, 1, TN)
    gsum, gcnt = pl.pallas_call(
        _k10, grid=(GN,),
        in_specs=[nblk, pl.BlockSpec((1, 1, TN), lambda i: (i, 0, 0)),
                  _full((8, H))],
        out_specs=(_full((G, H)), _full((G, 8))),
        out_shape=(sds((G, H), f32), sds((G, 8), f32)),
    )(hn2, batch3, _rows(w2a, z2a))

    W2g = jnp.zeros((H, 128), f32).at[:, :OUT].set(p['gW2'].T)
    b2g = jnp.zeros((8, 128), f32).at[0, :OUT].set(p['gb2'])
    outp = pl.pallas_call(
        _k11, grid=(1,),
        in_specs=[_full((G, H)), _full((G, 8)), _full((H, H)),
                  _full((8, H)), _full((H, 128)), _full((8, 128))],
        out_specs=_full((G, 128)),
        out_shape=sds((G, 128), f32),
    )(gsum, gcnt, p['gW1'].T, _rows(p['gb1'], p['gbn1_g'], p['gbn1_b']),
      W2g, b2g)
    return outp[:, :OUT]


# speedup vs baseline: 1.8237x; 1.0137x over previous
import functools

import jax
import jax.numpy as jnp
from jax import lax
from jax.experimental import pallas as pl
from jax.experimental.pallas import tpu as pltpu, tpu_sc as plsc

N = 50000
E = 800000
F = 74
FP = 128
H = 256
G = 512
OUT = 2
EPS = 1e-5

TE = 3200
TN = 2000
GE = E // TE
GN = N // TN

NC, NS = 2, 16
NW = NC * NS
EW = E // NW
GCH = 128
RNG = 2048
NRANGES = 26
NPAD = RNG * NRANGES
RPAD = 16
SCH = 2000
ECT = E // NS
DK = 128

_mesh = plsc.VectorSubcoreMesh(core_axis_name="c", subcore_axis_name="s")


@functools.partial(
    pl.kernel, mesh=_mesh,
    out_type=(jax.ShapeDtypeStruct((E, FP), jnp.float32),
              jax.ShapeDtypeStruct((E, FP), jnp.float32)),
    scratch_types=[pltpu.VMEM((GCH,), jnp.int32),
                   pltpu.VMEM((GCH,), jnp.int32),
                   pltpu.VMEM((GCH, FP), jnp.float32),
                   pltpu.VMEM((GCH, FP), jnp.float32),
                   pltpu.SemaphoreType.DMA,
                   pltpu.SemaphoreType.DMA],
)
def _sc_gather(xp_hbm, row_hbm, col_hbm, ar_hbm, ac_hbm,
               idx_a, idx_b, buf_a, buf_b, sem_a, sem_b):
    wid = lax.axis_index("s") * NC + lax.axis_index("c")
    base = wid * EW
    NFULL = EW // GCH
    TAIL = EW - NFULL * GCH

    def step(i, _):
        off = base + i * GCH
        pltpu.sync_copy(row_hbm.at[pl.ds(off, GCH)], idx_a)
        pltpu.sync_copy(col_hbm.at[pl.ds(off, GCH)], idx_b)
        cp_a = pltpu.async_copy(xp_hbm.at[idx_a], buf_a, sem_a)
        cp_b = pltpu.async_copy(xp_hbm.at[idx_b], buf_b, sem_b)
        cp_a.wait()
        pltpu.sync_copy(buf_a, ar_hbm.at[pl.ds(off, GCH)])
        cp_b.wait()
        pltpu.sync_copy(buf_b, ac_hbm.at[pl.ds(off, GCH)])
        return _

    lax.fori_loop(0, NFULL, step, 0)

    if TAIL:
        off = base + NFULL * GCH
        pltpu.sync_copy(row_hbm.at[pl.ds(off, TAIL)],
                        idx_a.at[pl.ds(0, TAIL)])
        pltpu.sync_copy(col_hbm.at[pl.ds(off, TAIL)],
                        idx_b.at[pl.ds(0, TAIL)])
        cp_a = pltpu.async_copy(xp_hbm.at[idx_a.at[pl.ds(0, TAIL)]],
                                buf_a.at[pl.ds(0, TAIL)], sem_a)
        cp_b = pltpu.async_copy(xp_hbm.at[idx_b.at[pl.ds(0, TAIL)]],
                                buf_b.at[pl.ds(0, TAIL)], sem_b)
        cp_a.wait()
        pltpu.sync_copy(buf_a.at[pl.ds(0, TAIL)],
                        ar_hbm.at[pl.ds(off, TAIL)])
        cp_b.wait()
        pltpu.sync_copy(buf_b.at[pl.ds(0, TAIL)],
                        ac_hbm.at[pl.ds(off, TAIL)])


@functools.partial(
    pl.kernel, mesh=_mesh,
    out_type=(jax.ShapeDtypeStruct((NPAD, H), jnp.float32),
              jax.ShapeDtypeStruct((NPAD, 16), jnp.float32)),
    scratch_types=[pltpu.VMEM((SCH,), jnp.int32),
                   pltpu.VMEM((SCH + 160,), jnp.int32),
                   pltpu.VMEM((SCH + 160,), jnp.int32),
                   pltpu.VMEM((DK,), jnp.int32),
                   pltpu.VMEM((DK,), jnp.int32),
                   pltpu.VMEM((DK, H), jnp.float32),
                   pltpu.VMEM((DK, H), jnp.float32),
                   pltpu.VMEM((16, H), jnp.float32),
                   pltpu.VMEM(((RNG + RPAD) // NS, 16), jnp.float32),
                   pltpu.VMEM((DK, 16), jnp.float32),
                   pltpu.VMEM_SHARED((RNG + RPAD, H), jnp.float32),
                   pltpu.VMEM_SHARED((RNG + RPAD, 16), jnp.float32),
                   pltpu.SemaphoreType.DMA,
                   pltpu.SemaphoreType.DMA],
    compiler_params=pltpu.CompilerParams(use_tc_tiling_on_sc=False,
                                         needs_layout_passes=False),
)
def _sc_scatter(g2_hbm, col_hbm, z_hbm, z16_hbm, on_hbm,
                nsum_hbm, ncnt_hbm,
                cbuf, midx, mloc, loc_a, loc_b, dbuf_a, dbuf_b,
                zbuf, z16v, ones_v, acc, cnta, sem_a, sem_b):
    c = lax.axis_index("c")
    s = lax.axis_index("s")
    iota = lax.iota(jnp.int32, 16)
    tbase = s * ECT
    pltpu.sync_copy(z_hbm, zbuf)
    pltpu.sync_copy(z16_hbm, z16v)
    pltpu.sync_copy(on_hbm, ones_v)

    def _drain_one(k0, loc, dbuf, sem):
        for j in range(DK // 16):
            loc[pl.ds(j * 16, 16)] = mloc[pl.ds(k0 * DK + j * 16, 16)]
        return pltpu.async_copy(g2_hbm.at[midx.at[pl.ds(k0 * DK, DK)]],
                                dbuf, sem)

    def _add_one(loc, dbuf):
        pltpu.sync_copy(dbuf, acc.at[loc], add=True)
        pltpu.sync_copy(ones_v, cnta.at[loc], add=True)

    def range_step(r_i, carry_r):
        rng = 2 * r_i + c
        lo = rng * RNG
        def zstep(k, _z):
            pltpu.sync_copy(zbuf,
                            acc.at[pl.ds(s * (RNG // NS) + k * 16, 16)])
            return _z
        lax.fori_loop(0, RNG // NS // 16, zstep, 0)
        pltpu.sync_copy(z16v, cnta.at[pl.ds(s * ((RNG + RPAD) // NS),
                                            (RNG + RPAD) // NS)])
        plsc.subcore_barrier()

        def chunk_step(ch, cur_in):
            ebase = tbase + ch * SCH
            pltpu.sync_copy(col_hbm.at[pl.ds(ebase, SCH)], cbuf)

            def scan_step(v, cur):
                cols = cbuf[pl.ds(v * 16, 16)]
                m = (cols >= lo) & (cols < lo + RNG)
                pre = plsc.cumsum(m.astype(jnp.int32))
                eid = ebase + v * 16 + iota
                dst = jnp.where(m, cur + pre - 1, SCH + 128 + iota)
                plsc.store_scatter(midx, [dst], eid)
                plsc.store_scatter(mloc, [dst], cols - lo)
                return cur + pre[15]

            cur = lax.fori_loop(0, SCH // 16, scan_step, cur_in)
            n128 = cur // DK

            def pair_step(p, _d):
                cp_a = _drain_one(2 * p, loc_a, dbuf_a, sem_a)
                cp_b = _drain_one(2 * p + 1, loc_b, dbuf_b, sem_b)
                cp_a.wait()
                _add_one(loc_a, dbuf_a)
                cp_b.wait()
                _add_one(loc_b, dbuf_b)
                return _d

            lax.fori_loop(0, n128 // 2, pair_step, 0)

            @pl.when(n128 % 2 == 1)
            def _():
                _drain_one(n128 - 1, loc_a, dbuf_a, sem_a).wait()
                _add_one(loc_a, dbuf_a)

            rem = cur - n128 * DK
            for j in range(DK // 16):
                tmp_i = midx[pl.ds(n128 * DK + j * 16, 16)]
                tmp_l = mloc[pl.ds(n128 * DK + j * 16, 16)]
                midx[pl.ds(j * 16, 16)] = tmp_i
                mloc[pl.ds(j * 16, 16)] = tmp_l
            return rem

        rem = lax.fori_loop(0, ECT // SCH, chunk_step, 0)

        for j in range(DK // 16):
            midx[pl.ds(rem + j * 16, 16)] = jnp.zeros((16,), jnp.int32)
            mloc[pl.ds(rem + j * 16, 16)] = jnp.full((16,), RNG, jnp.int32)

        @pl.when(rem > 0)
        def _():
            _drain_one(0, loc_a, dbuf_a, sem_a).wait()
            _add_one(loc_a, dbuf_a)

        plsc.subcore_barrier()

        out0 = rng * RNG + s * (RNG // NS)
        pltpu.sync_copy(acc.at[pl.ds(s * (RNG // NS), RNG // NS)],
                        nsum_hbm.at[pl.ds(out0, RNG // NS)])
        pltpu.sync_copy(cnta.at[pl.ds(s * (RNG // NS), RNG // NS)],
                        ncnt_hbm.at[pl.ds(out0, RNG // NS)])
        plsc.subcore_barrier()
        return carry_r

    lax.fori_loop(0, NRANGES // NC, range_step, 0)


def _full(shape):
    nd = len(shape)
    return pl.BlockSpec(shape, lambda i, _n=nd: (0,) * _n)


def _stats_of(h, stats_ref, i):
    @pl.when(i == 0)
    def _():
        stats_ref[...] = jnp.zeros_like(stats_ref)
    stats_ref[0:1, :] += jnp.sum(h, axis=0, keepdims=True)
    stats_ref[1:2, :] += jnp.sum(h * h, axis=0, keepdims=True)


def _affine(stats, n, g, b):
    m = stats[0] / n
    v = stats[1] / n - m * m
    s = g / jnp.sqrt(v + EPS)
    return s, b - m * s


def _k1(x_ref, stats_ref):
    _stats_of(x_ref[...], stats_ref, pl.program_id(0))


def _k3(ar_ref, ac_ref, w1r_ref, w1c_ref, p_ref, stats_ref):
    h1 = (jnp.dot(ar_ref[...], w1r_ref[...],
                  preferred_element_type=jnp.float32)
          + jnp.dot(ac_ref[...], w1c_ref[...],
                    preferred_element_type=jnp.float32)
          + p_ref[0:1, :])
    _stats_of(h1, stats_ref, pl.program_id(0))


def _k4(ar_ref, ac_ref, w1r_ref, w1c_ref, w2_ref, p_ref, h2_ref, stats_ref):
    h1 = (jnp.dot(ar_ref[...], w1r_ref[...],
                  preferred_element_type=jnp.float32)
          + jnp.dot(ac_ref[...], w1c_ref[...],
                    preferred_element_type=jnp.float32)
          + p_ref[0:1, :])
    e1 = jnp.maximum(h1 * p_ref[1:2, :] + p_ref[2:3, :], 0.0)
    h2 = jnp.dot(e1, w2_ref[...],
                 preferred_element_type=jnp.float32) + p_ref[3:4, :]
    h2_ref[...] = h2
    _stats_of(h2, stats_ref, pl.program_id(0))


def _k5(h2_ref, ar_ref, wx_ref, we_ref, p_ref, g1_ref, stats_ref):
    e = h2_ref[...] * p_ref[0:1, :] + p_ref[1:2, :]
    g1 = (jnp.dot(ar_ref[...], wx_ref[...],
                  preferred_element_type=jnp.float32)
          + jnp.dot(e, we_ref[...], preferred_element_type=jnp.float32)
          + p_ref[2:3, :])
    g1_ref[...] = g1
    _stats_of(g1, stats_ref, pl.program_id(0))


def _k6(g1_ref, w_ref, p_ref, g2_ref, stats_ref):
    f1 = jnp.maximum(g1_ref[...] * p_ref[0:1, :] + p_ref[1:2, :], 0.0)
    g2 = jnp.dot(f1, w_ref[...],
                 preferred_element_type=jnp.float32) + p_ref[2:3, :]
    g2_ref[...] = g2
    _stats_of(g2, stats_ref, pl.program_id(0))


def _k8(x_ref, nsum_ref, ncnt_ref, wx_ref, wa_ref, p_ref, hn1_ref, stats_ref):
    cnt = ncnt_ref[...][:, 0:1]
    mean = nsum_ref[...] / jnp.maximum(cnt, 1.0)
    agg = jnp.where(cnt > 0.0, mean * p_ref[0:1, :] + p_ref[1:2, :], 0.0)
    hn1 = (jnp.dot(x_ref[...], wx_ref[...],
                   preferred_element_type=jnp.float32)
           + jnp.dot(agg, wa_ref[...], preferred_element_type=jnp.float32)
           + p_ref[2:3, :])
    hn1_ref[...] = hn1
    _stats_of(hn1, stats_ref, pl.program_id(0))


_k9 = _k6


def _k10(hn2_ref, b_ref, p_ref, gsum_ref, gcnt_ref):
    i = pl.program_id(0)
    xn = hn2_ref[...] * p_ref[0:1, :] + p_ref[1:2, :]
    b = b_ref[0]
    onehot = (b.reshape(TN, 1)
              == lax.broadcasted_iota(jnp.int32, (TN, G), 1)
              ).astype(jnp.float32)

    @pl.when(i == 0)
    def _():
        gsum_ref[...] = jnp.zeros_like(gsum_ref)
        gcnt_ref[...] = jnp.zeros_like(gcnt_ref)

    gsum_ref[...] += lax.dot_general(onehot, xn, (((0,), (0,)), ((), ())),
                                     preferred_element_type=jnp.float32)
    gcnt_ref[...] += lax.dot_general(onehot, jnp.ones((TN, 8), jnp.float32),
                                     (((0,), (0,)), ((), ())),
                                     preferred_element_type=jnp.float32)


def _k11(gsum_ref, gcnt_ref, w1_ref, p_ref, w2_ref, b2_ref, out_ref):
    cnt = gcnt_ref[...][:, 0:1]
    u = gsum_ref[...] / jnp.maximum(cnt, 1.0)
    uu = jnp.dot(u, w1_ref[...],
                 preferred_element_type=jnp.float32) + p_ref[0:1, :]
    m = jnp.mean(uu, axis=0, keepdims=True)
    v = jnp.mean(uu * uu, axis=0, keepdims=True) - m * m
    uu = (uu - m) / jnp.sqrt(v + EPS) * p_ref[1:2, :] + p_ref[2:3, :]
    uu = jnp.maximum(uu, 0.0)
    out_ref[...] = jnp.dot(uu, w2_ref[...],
                           preferred_element_type=jnp.float32) + b2_ref[0:1, :]


def _rows(*vs):
    w = vs[0].shape[-1]
    out = jnp.zeros((8, w), jnp.float32)
    for i, v in enumerate(vs):
        out = out.at[i].set(v)
    return out


def kernel(x, params, edge_index, batch):
    p = params
    row = edge_index[0]
    col = edge_index[1]
    x_pad = jnp.pad(x, ((0, 0), (0, FP - F)))

    f32 = jnp.float32
    sds = jax.ShapeDtypeStruct

    xstats = pl.pallas_call(
        _k1, grid=(GN,),
        in_specs=[pl.BlockSpec((TN, FP), lambda i: (i, 0))],
        out_specs=_full((8, FP)),
        out_shape=sds((8, FP), f32),
    )(x_pad)
    s0, t0 = _affine(xstats[:, :F], N, p['bn0_g'], p['bn0_b'])
    s0p = jnp.pad(s0, (0, FP - F))
    t0p = jnp.pad(t0, (0, FP - F))

    W1r = jnp.pad((p['eW1'][:, :F] * s0[None, :]).T, ((0, FP - F), (0, 0)))
    W1c = jnp.pad((p['eW1'][:, F:] * s0[None, :]).T, ((0, FP - F), (0, 0)))
    b1 = p['eb1'] + (p['eW1'][:, :F] + p['eW1'][:, F:]) @ t0
    W2e = p['eW2'].T
    W1nx = jnp.pad((p['n1W1'][:, :F] * s0[None, :]).T, ((0, FP - F), (0, 0)))
    W1ne = p['n1W1'][:, F:].T
    c1 = p['n1b1'] + p['n1W1'][:, :F] @ t0
    W2n = p['n1W2'].T
    W2nx = jnp.pad((p['n2W1'][:, :F] * s0[None, :]).T, ((0, FP - F), (0, 0)))
    W2na = p['n2W1'][:, F:].T
    d1 = p['n2b1'] + p['n2W1'][:, :F] @ t0
    W2n2 = p['n2W2'].T

    ar, ac = _sc_gather(x_pad, row, col)

    eblk = pl.BlockSpec((TE, FP), lambda i: (i, 0))
    hblk = pl.BlockSpec((TE, H), lambda i: (i, 0))

    st1 = pl.pallas_call(
        _k3, grid=(GE,),
        in_specs=[eblk, eblk, _full((FP, H)), _full((FP, H)), _full((8, H))],
        out_specs=_full((8, H)),
        out_shape=sds((8, H), f32),
    )(ar, ac, W1r, W1c, _rows(b1))
    s1, t1 = _affine(st1, E, p['ebn1_g'], p['ebn1_b'])

    h2, st2 = pl.pallas_call(
        _k4, grid=(GE,),
        in_specs=[eblk, eblk, _full((FP, H)), _full((FP, H)),
                  _full((H, H)), _full((8, H))],
        out_specs=(hblk, _full((8, H))),
        out_shape=(sds((E, H), f32), sds((8, H), f32)),
    )(ar, ac, W1r, W1c, W2e, _rows(b1, s1, t1, p['eb2']))
    s2, t2 = _affine(st2, E, p['ebn2_g'], p['ebn2_b'])

    g1, st3 = pl.pallas_call(
        _k5, grid=(GE,),
        in_specs=[hblk, eblk, _full((FP, H)), _full((H, H)), _full((8, H))],
        out_specs=(hblk, _full((8, H))),
        out_shape=(sds((E, H), f32), sds((8, H), f32)),
    )(h2, ar, W1nx, W1ne, _rows(s2, t2, c1))
    u1, v1 = _affine(st3, E, p['n1bn1_g'], p['n1bn1_b'])

    g2, st4 = pl.pallas_call(
        _k6, grid=(GE,),
        in_specs=[hblk, _full((H, H)), _full((8, H))],
        out_specs=(hblk, _full((8, H))),
        out_shape=(sds((E, H), f32), sds((8, H), f32)),
    )(g1, W2n, _rows(u1, v1, p['n1b2']))
    u2, v2 = _affine(st4, E, p['n1bn2_g'], p['n1bn2_b'])

    zeros64 = jnp.zeros((16, H), f32)
    zeros16 = jnp.zeros(((RNG + RPAD) // NS, 16), f32)
    ones16 = jnp.ones((DK, 16), f32)
    nsum, ncnt16 = _sc_scatter(g2, col, zeros64, zeros16, ones16)

    nblk = pl.BlockSpec((TN, H), lambda i: (i, 0))

    hn1, st5 = pl.pallas_call(
        _k8, grid=(GN,),
        in_specs=[pl.BlockSpec((TN, FP), lambda i: (i, 0)), nblk,
                  pl.BlockSpec((TN, 16), lambda i: (i, 0)),
                  _full((FP, H)), _full((H, H)), _full((8, H))],
        out_specs=(nblk, _full((8, H))),
        out_shape=(sds((N, H), f32), sds((8, H), f32)),
    )(x_pad, nsum, ncnt16, W2nx, W2na, _rows(u2, v2, d1))
    w1a, z1a = _affine(st5, N, p['n2bn1_g'], p['n2bn1_b'])

    hn2, st6 = pl.pallas_call(
        _k9, grid=(GN,),
        in_specs=[nblk, _full((H, H)), _full((8, H))],
        out_specs=(nblk, _full((8, H))),
        out_shape=(sds((N, H), f32), sds((8, H), f32)),
    )(hn1, W2n2, _rows(w1a, z1a, p['n2b2']))
    w2a, z2a = _affine(st6, N, p['n2bn2_g'], p['n2bn2_b'])

    batch3 = batch.reshape(GN, 1, TN)
    gsum, gcnt = pl.pallas_call(
        _k10, grid=(GN,),
        in_specs=[nblk, pl.BlockSpec((1, 1, TN), lambda i: (i, 0, 0)),
                  _full((8, H))],
        out_specs=(_full((G, H)), _full((G, 8))),
        out_shape=(sds((G, H), f32), sds((G, 8), f32)),
    )(hn2, batch3, _rows(w2a, z2a))

    W2g = jnp.zeros((H, 128), f32).at[:, :OUT].set(p['gW2'].T)
    b2g = jnp.zeros((8, 128), f32).at[0, :OUT].set(p['gb2'])
    outp = pl.pallas_call(
        _k11, grid=(1,),
        in_specs=[_full((G, H)), _full((G, 8)), _full((H, H)),
                  _full((8, H)), _full((H, 128)), _full((8, 128))],
        out_specs=_full((G, 128)),
        out_shape=sds((G, 128), f32),
    )(gsum, gcnt, p['gW1'].T, _rows(p['gb1'], p['gbn1_g'], p['gbn1_b']),
      W2g, b2g)
    return outp[:, :OUT]
